# baseline (speedup 1.0000x reference)
import functools

import jax
import jax.numpy as jnp
from jax import lax
from jax.experimental import pallas as pl
from jax.experimental.pallas import tpu as pltpu
from jax.experimental.pallas import tpu_sc as plsc

NC = 2
NS = 16
LANES = 16


def _largest_div(total, hi, step):
    for c in range(hi, step - 1, -step):
        if total % c == 0:
            return c
    return None


def _project(x2, W, b2):
    M, Cin = x2.shape
    Cout = W.shape[1]
    BM = _largest_div(M, 2048, 8) or M

    def body(x_ref, w_ref, b_ref, o_ref):
        o_ref[...] = (
            jnp.dot(x_ref[...], w_ref[...], preferred_element_type=jnp.float32)
            + b_ref[...]
        )

    return pl.pallas_call(
        body,
        grid=(M // BM,),
        in_specs=[
            pl.BlockSpec((BM, Cin), lambda i: (i, 0)),
            pl.BlockSpec((Cin, Cout), lambda i: (0, 0)),
            pl.BlockSpec((1, Cout), lambda i: (0, 0)),
        ],
        out_specs=pl.BlockSpec((BM, Cout), lambda i: (i, 0)),
        out_shape=jax.ShapeDtypeStruct((M, Cout), jnp.float32),
    )(x2, W, b2)


def _aggregate(h, src_flat, dst_flat, B, N, E, C):
    assert B % NC == 0 and C % LANES == 0
    ROUNDS = B // NC
    CH = 128
    NCHUNK = src_flat.shape[0] // (B * NS)
    NACC = N + 8
    RCH = _largest_div(N, 128, 16)
    assert RCH is not None
    NROWCH = N // RCH
    ITER_R = -(-NROWCH // NS)
    NCC = C // LANES
    BUF = max(CH, RCH)
    IB = _largest_div(NCHUNK, 40, 8) or NCHUNK
    NBLK = NCHUNK // IB
    assert IB % 2 == 0

    mesh = plsc.VectorSubcoreMesh(core_axis_name="c", subcore_axis_name="s")

    @functools.partial(
        pl.kernel,
        mesh=mesh,
        out_type=jax.ShapeDtypeStruct((B * N, C), jnp.float32),
        scratch_types=[
            pltpu.VMEM((IB, CH), jnp.int32),
            pltpu.VMEM((IB, CH), jnp.int32),
            pltpu.VMEM((2 * CH, C), jnp.float32),
            pltpu.VMEM((BUF,), jnp.float32),
            pltpu.VMEM_SHARED((NACC, C), jnp.float32),
            pltpu.VMEM_SHARED((NACC,), jnp.float32),
            pltpu.SemaphoreType.DMA,
            pltpu.SemaphoreType.DMA,
        ],
    )
    def agg(h_hbm, src_hbm, dst_hbm, out_hbm,
            idx_s, idx_d, rowbuf, smallbuf,
            acc_sh, deg_sh, semA, semB):
        c = lax.axis_index("c")
        s = lax.axis_index("s")

        one16 = jnp.full((LANES,), 1.0, jnp.float32)
        zero16 = jnp.zeros((LANES,), jnp.float32)

        def fill_small(val):
            def fbody(i, _):
                smallbuf[pl.ds(i * LANES, LANES)] = val
                return 0
            lax.fori_loop(0, BUF // LANES, fbody, 0)

        for r in range(ROUNDS):
            g = r * NC + c

            fill_small(zero16)

            def zbody(i, _):
                for cc in range(NCC):
                    rowbuf[i, pl.ds(cc * LANES, LANES)] = zero16
                return 0
            lax.fori_loop(0, BUF, zbody, 0)
            for i in range(ITER_R):
                k = i * NS + s

                @pl.when(k < NROWCH)
                def _():
                    rb = k * RCH
                    pltpu.sync_copy(rowbuf.at[pl.ds(0, RCH)], acc_sh.at[pl.ds(rb, RCH)])
                    pltpu.sync_copy(smallbuf.at[pl.ds(0, RCH)], deg_sh.at[pl.ds(rb, RCH)])
            fill_small(one16)
            plsc.subcore_barrier()

            H = CH // 2
            ones_ch = smallbuf.at[pl.ds(0, CH)]

            def start_gather(j, sb, sem):
                pltpu.async_copy(h_hbm.at[idx_s.at[j, pl.ds(0, H)]],
                                 rowbuf.at[pl.ds(sb, H)], sem)
                pltpu.async_copy(h_hbm.at[idx_s.at[j, pl.ds(H, H)]],
                                 rowbuf.at[pl.ds(sb + H, H)], sem)

            def finish_chunk(j, sb, sem, start_j):
                pltpu.make_async_copy(h_hbm.at[idx_s.at[j]],
                                      rowbuf.at[pl.ds(sb, CH)], sem).wait()
                pltpu.sync_copy(rowbuf.at[pl.ds(sb, CH)],
                                acc_sh.at[idx_d.at[j]], add=True)
                pltpu.sync_copy(ones_ch, deg_sh.at[idx_d.at[j]], add=True)

                @pl.when(start_j < IB)
                def _():
                    start_gather(start_j, sb, sem)

            def blkbody(blk, _):
                rowb = (g * NS + s) * NCHUNK + blk * IB
                pltpu.sync_copy(src_hbm.at[pl.ds(rowb, IB)], idx_s)
                pltpu.sync_copy(dst_hbm.at[pl.ds(rowb, IB)], idx_d)
                start_gather(0, 0, semA)
                start_gather(1, CH, semB)

                def pair(p, _):
                    j0 = 2 * p
                    finish_chunk(j0, 0, semA, j0 + 2)
                    finish_chunk(j0 + 1, CH, semB, j0 + 3)
                    return 0
                lax.fori_loop(0, IB // 2, pair, 0)
                return 0
            lax.fori_loop(0, NBLK, blkbody, 0)
            plsc.subcore_barrier()

            for i in range(ITER_R):
                k = i * NS + s

                @pl.when(k < NROWCH)
                def _():
                    rb = k * RCH
                    pltpu.sync_copy(acc_sh.at[pl.ds(rb, RCH)], rowbuf.at[pl.ds(0, RCH)])
                    pltpu.sync_copy(deg_sh.at[pl.ds(rb, RCH)], smallbuf.at[pl.ds(0, RCH)])

                    def rbody(q, _):
                        dvec = smallbuf[pl.ds(q * LANES, LANES)]
                        rec = one16 / jnp.maximum(dvec, one16)
                        for rr in range(LANES):
                            r2 = q * LANES + rr
                            rec16 = jnp.broadcast_to(rec[rr], (LANES,))
                            for cc in range(NCC):
                                sl = pl.ds(cc * LANES, LANES)
                                rowbuf[r2, sl] = rowbuf[r2, sl] * rec16
                        return 0
                    lax.fori_loop(0, RCH // LANES, rbody, 0)
                    pltpu.sync_copy(rowbuf.at[pl.ds(0, RCH)], out_hbm.at[pl.ds(g * N + rb, RCH)])
            plsc.subcore_barrier()

    return agg(h, src_flat, dst_flat)


def kernel(x, edge_index, W, b):
    B, N, Cin = x.shape
    Cout = W.shape[1]
    E = edge_index.shape[2]

    h = _project(x.reshape(B * N, Cin), W, b.reshape(1, Cout))

    offs = (jnp.arange(B, dtype=jnp.int32) * N)[:, None]
    src = (edge_index[:, 1, :] + offs).reshape(B * E)
    dst = edge_index[:, 0, :].reshape(B * E)

    CH = 128
    EPT = E // NS
    NCHUNK = (-(-EPT // CH) + 7) // 8 * 8
    EPT_PAD = NCHUNK * CH
    src3 = src.reshape(B, NS, EPT)
    dst3 = dst.reshape(B, NS, EPT)
    pad = ((0, 0), (0, 0), (0, EPT_PAD - EPT))
    src2d = jnp.pad(src3, pad, constant_values=0).reshape(B * NS * NCHUNK, CH)
    dst2d = jnp.pad(dst3, pad, constant_values=N).reshape(B * NS * NCHUNK, CH)

    out = _aggregate(h, src2d, dst2d, B=B, N=N, E=E, C=Cout)
    return out.reshape(B, N, Cout)


# speedup vs baseline: 2.5318x; 2.5318x over previous
import functools

import jax
import jax.numpy as jnp
from jax import lax
from jax.experimental import pallas as pl
from jax.experimental.pallas import tpu as pltpu
from jax.experimental.pallas import tpu_sc as plsc

NC = 2
NS = 16
LANES = 16


def _largest_div(total, hi, step):
    for c in range(hi, step - 1, -step):
        if total % c == 0:
            return c
    return None


def _project(x2, W, b2):
    M, Cin = x2.shape
    Cout = W.shape[1]
    BM = _largest_div(M, 2048, 8) or M

    def body(x_ref, w_ref, b_ref, o_ref):
        o_ref[...] = (
            jnp.dot(x_ref[...], w_ref[...], preferred_element_type=jnp.float32)
            + b_ref[...]
        )

    return pl.pallas_call(
        body,
        grid=(M // BM,),
        in_specs=[
            pl.BlockSpec((BM, Cin), lambda i: (i, 0)),
            pl.BlockSpec((Cin, Cout), lambda i: (0, 0)),
            pl.BlockSpec((1, Cout), lambda i: (0, 0)),
        ],
        out_specs=pl.BlockSpec((BM, Cout), lambda i: (i, 0)),
        out_shape=jax.ShapeDtypeStruct((M, Cout), jnp.float32),
    )(x2, W, b2)


def _aggregate(h, src_flat, dst_flat, B, N, E, C):
    assert B % NC == 0 and E % NS == 0 and C % LANES == 0
    ROUNDS = B // NC
    EPT = E // NS
    CH = _largest_div(EPT, 80, 8)
    assert CH is not None
    NCHUNK = EPT // CH
    assert NCHUNK % 2 == 0
    RCH = _largest_div(N, 128, 16)
    assert RCH is not None
    NROWCH = N // RCH
    ITER_R = -(-NROWCH // NS)
    NCC = C // LANES
    BUF = max(CH, RCH)

    mesh = plsc.VectorSubcoreMesh(core_axis_name="c", subcore_axis_name="s")

    @functools.partial(
        pl.kernel,
        mesh=mesh,
        out_type=jax.ShapeDtypeStruct((B * N, C), jnp.float32),
        scratch_types=[
            pltpu.VMEM((CH,), jnp.int32),
            pltpu.VMEM((CH,), jnp.int32),
            pltpu.VMEM((CH,), jnp.int32),
            pltpu.VMEM((CH,), jnp.int32),
            pltpu.VMEM((BUF, C), jnp.float32),
            pltpu.VMEM((CH, C), jnp.float32),
            pltpu.VMEM((BUF,), jnp.float32),
            pltpu.VMEM_SHARED((N, C), jnp.float32),
            pltpu.VMEM_SHARED((N,), jnp.float32),
            pltpu.SemaphoreType.DMA,
            pltpu.SemaphoreType.DMA,
            pltpu.SemaphoreType.DMA,
            pltpu.SemaphoreType.DMA,
        ],
    )
    def agg(h_hbm, src_hbm, dst_hbm, out_hbm,
            ixs0, ixd0, ixs1, ixd1, row0, row1, smallbuf,
            acc_sh, deg_sh, semI0, semI1, semG0, semG1):
        c = lax.axis_index("c")
        s = lax.axis_index("s")

        one16 = jnp.full((LANES,), 1.0, jnp.float32)
        zero16 = jnp.zeros((LANES,), jnp.float32)
        row0c = row0.at[pl.ds(0, CH)]
        ones_ch = smallbuf.at[pl.ds(0, CH)]

        def fill_small(val):
            def fbody(i, _):
                smallbuf[pl.ds(i * LANES, LANES)] = val
                return 0
            lax.fori_loop(0, BUF // LANES, fbody, 0)

        for r in range(ROUNDS):
            g = r * NC + c

            fill_small(zero16)

            def zbody(i, _):
                for cc in range(NCC):
                    row0[i, pl.ds(cc * LANES, LANES)] = zero16
                return 0
            lax.fori_loop(0, BUF, zbody, 0)
            for i in range(ITER_R):
                k = i * NS + s

                @pl.when(k < NROWCH)
                def _():
                    rb = k * RCH
                    pltpu.sync_copy(row0.at[pl.ds(0, RCH)], acc_sh.at[pl.ds(rb, RCH)])
                    pltpu.sync_copy(smallbuf.at[pl.ds(0, RCH)], deg_sh.at[pl.ds(rb, RCH)])
            fill_small(one16)
            plsc.subcore_barrier()

            ebase = g * E + s * EPT

            def start_idx(j, ixs, ixd, semI):
                e0 = ebase + j * CH
                pltpu.async_copy(src_hbm.at[pl.ds(e0, CH)], ixs, semI)
                pltpu.async_copy(dst_hbm.at[pl.ds(e0, CH)], ixd, semI)

            def wait_idx(ixs, ixd, semI):
                pltpu.make_async_copy(src_hbm.at[pl.ds(ebase, CH)], ixs, semI).wait()
                pltpu.make_async_copy(dst_hbm.at[pl.ds(ebase, CH)], ixd, semI).wait()

            start_idx(0, ixs0, ixd0, semI0)
            wait_idx(ixs0, ixd0, semI0)
            pltpu.async_copy(h_hbm.at[ixs0], row0c, semG0)
            start_idx(1, ixs1, ixd1, semI1)
            wait_idx(ixs1, ixd1, semI1)

            def pair(p, _):
                j0 = 2 * p
                j1 = j0 + 1
                pltpu.make_async_copy(h_hbm.at[ixs0], row0c, semG0).wait()
                pltpu.async_copy(h_hbm.at[ixs1], row1, semG1)
                pltpu.sync_copy(row0c, acc_sh.at[ixd0], add=True)
                pltpu.sync_copy(ones_ch, deg_sh.at[ixd0], add=True)

                @pl.when(j0 + 2 < NCHUNK)
                def _():
                    start_idx(j0 + 2, ixs0, ixd0, semI0)
                    wait_idx(ixs0, ixd0, semI0)
                pltpu.make_async_copy(h_hbm.at[ixs1], row1, semG1).wait()

                @pl.when(j0 + 2 < NCHUNK)
                def _():
                    pltpu.async_copy(h_hbm.at[ixs0], row0c, semG0)
                pltpu.sync_copy(row1, acc_sh.at[ixd1], add=True)
                pltpu.sync_copy(ones_ch, deg_sh.at[ixd1], add=True)

                @pl.when(j1 + 2 < NCHUNK)
                def _():
                    start_idx(j1 + 2, ixs1, ixd1, semI1)
                    wait_idx(ixs1, ixd1, semI1)
                return 0
            lax.fori_loop(0, NCHUNK // 2, pair, 0)
            plsc.subcore_barrier()

            for i in range(ITER_R):
                k = i * NS + s

                @pl.when(k < NROWCH)
                def _():
                    rb = k * RCH
                    pltpu.sync_copy(acc_sh.at[pl.ds(rb, RCH)], row0.at[pl.ds(0, RCH)])
                    pltpu.sync_copy(deg_sh.at[pl.ds(rb, RCH)], smallbuf.at[pl.ds(0, RCH)])

                    def rbody(q, _):
                        dvec = smallbuf[pl.ds(q * LANES, LANES)]
                        rec = one16 / jnp.maximum(dvec, one16)
                        for rr in range(LANES):
                            r2 = q * LANES + rr
                            rec16 = jnp.broadcast_to(rec[rr], (LANES,))
                            for cc in range(NCC):
                                sl = pl.ds(cc * LANES, LANES)
                                row0[r2, sl] = row0[r2, sl] * rec16
                        return 0
                    lax.fori_loop(0, RCH // LANES, rbody, 0)
                    pltpu.sync_copy(row0.at[pl.ds(0, RCH)], out_hbm.at[pl.ds(g * N + rb, RCH)])
            plsc.subcore_barrier()

    return agg(h, src_flat, dst_flat)


def kernel(x, edge_index, W, b):
    B, N, Cin = x.shape
    Cout = W.shape[1]
    E = edge_index.shape[2]

    h = _project(x.reshape(B * N, Cin), W, b.reshape(1, Cout))

    offs = (jnp.arange(B, dtype=jnp.int32) * N)[:, None]
    src = (edge_index[:, 1, :] + offs).reshape(B * E)
    dst = edge_index[:, 0, :].reshape(B * E)

    out = _aggregate(h, src, dst, B=B, N=N, E=E, C=Cout)
    return out.reshape(B, N, Cout)


# speedup vs baseline: 2.6316x; 1.0394x over previous
import functools

import jax
import jax.numpy as jnp
from jax import lax
from jax.experimental import pallas as pl
from jax.experimental.pallas import tpu as pltpu
from jax.experimental.pallas import tpu_sc as plsc

NC = 2
NS = 16
LANES = 16


def _largest_div(total, hi, step):
    for c in range(hi, step - 1, -step):
        if total % c == 0:
            return c
    return None


def _project(x2, W, b2):
    M, Cin = x2.shape
    Cout = W.shape[1]
    BM = _largest_div(M, 2048, 8) or M

    def body(x_ref, w_ref, b_ref, o_ref):
        o_ref[...] = (
            jnp.dot(x_ref[...], w_ref[...], preferred_element_type=jnp.float32)
            + b_ref[...]
        )

    return pl.pallas_call(
        body,
        grid=(M // BM,),
        in_specs=[
            pl.BlockSpec((BM, Cin), lambda i: (i, 0)),
            pl.BlockSpec((Cin, Cout), lambda i: (0, 0)),
            pl.BlockSpec((1, Cout), lambda i: (0, 0)),
        ],
        out_specs=pl.BlockSpec((BM, Cout), lambda i: (i, 0)),
        out_shape=jax.ShapeDtypeStruct((M, Cout), jnp.float32),
    )(x2, W, b2)


def _aggregate(h, src_flat, dst_flat, B, N, E, C):
    assert B % NC == 0 and E % NS == 0 and C % LANES == 0
    ROUNDS = B // NC
    EPT = E // NS
    CH = _largest_div(EPT, 80, 8)
    assert CH is not None
    NCHUNK = EPT // CH
    assert NCHUNK % 2 == 0
    RCH = _largest_div(N, 128, 16)
    assert RCH is not None
    NROWCH = N // RCH
    ITER_R = -(-NROWCH // NS)
    NCC = C // LANES
    BUF = max(CH, RCH)

    mesh = plsc.VectorSubcoreMesh(core_axis_name="c", subcore_axis_name="s")

    @functools.partial(
        pl.kernel,
        mesh=mesh,
        out_type=jax.ShapeDtypeStruct((B * N, C), jnp.float32),
        scratch_types=[
            pltpu.VMEM((CH,), jnp.int32),
            pltpu.VMEM((CH,), jnp.int32),
            pltpu.VMEM((CH,), jnp.int32),
            pltpu.VMEM((CH,), jnp.int32),
            pltpu.VMEM((BUF, C), jnp.float32),
            pltpu.VMEM((CH, C), jnp.float32),
            pltpu.VMEM((BUF,), jnp.float32),
            pltpu.VMEM_SHARED((N, C), jnp.float32),
            pltpu.VMEM_SHARED((N,), jnp.float32),
            pltpu.SemaphoreType.DMA,
            pltpu.SemaphoreType.DMA,
            pltpu.SemaphoreType.DMA,
            pltpu.SemaphoreType.DMA,
        ],
    )
    def agg(h_hbm, src_hbm, dst_hbm, out_hbm,
            ixs0, ixd0, ixs1, ixd1, row0, row1, smallbuf,
            acc_sh, deg_sh, semI0, semI1, semG0, semG1):
        c = lax.axis_index("c")
        s = lax.axis_index("s")

        one16 = jnp.full((LANES,), 1.0, jnp.float32)
        zero16 = jnp.zeros((LANES,), jnp.float32)
        row0c = row0.at[pl.ds(0, CH)]
        ones_ch = smallbuf.at[pl.ds(0, CH)]

        def fill_small(val):
            def fbody(i, _):
                smallbuf[pl.ds(i * LANES, LANES)] = val
                return 0
            lax.fori_loop(0, BUF // LANES, fbody, 0)

        for r in range(ROUNDS):
            g = r * NC + c

            fill_small(zero16)

            def zbody(i, _):
                for cc in range(NCC):
                    row0[i, pl.ds(cc * LANES, LANES)] = zero16
                return 0
            lax.fori_loop(0, BUF, zbody, 0)
            for i in range(ITER_R):
                k = i * NS + s

                @pl.when(k < NROWCH)
                def _():
                    rb = k * RCH
                    pltpu.sync_copy(row0.at[pl.ds(0, RCH)], acc_sh.at[pl.ds(rb, RCH)])
                    pltpu.sync_copy(smallbuf.at[pl.ds(0, RCH)], deg_sh.at[pl.ds(rb, RCH)])
            fill_small(one16)
            plsc.subcore_barrier()

            ebase = g * E + s * EPT

            def start_idx(j, ixs, ixd, semI):
                e0 = ebase + j * CH
                pltpu.async_copy(src_hbm.at[pl.ds(e0, CH)], ixs, semI)
                pltpu.async_copy(dst_hbm.at[pl.ds(e0, CH)], ixd, semI)

            def wait_idx(ixs, ixd, semI):
                pltpu.make_async_copy(src_hbm.at[pl.ds(ebase, CH)], ixs, semI).wait()
                pltpu.make_async_copy(dst_hbm.at[pl.ds(ebase, CH)], ixd, semI).wait()

            start_idx(0, ixs0, ixd0, semI0)
            wait_idx(ixs0, ixd0, semI0)
            pltpu.async_copy(h_hbm.at[ixs0], row0c, semG0)
            start_idx(1, ixs1, ixd1, semI1)
            wait_idx(ixs1, ixd1, semI1)

            def pair(p, _):
                j0 = 2 * p
                j1 = j0 + 1
                pltpu.async_copy(h_hbm.at[ixs1], row1, semG1)
                pltpu.make_async_copy(h_hbm.at[ixs0], row0c, semG0).wait()
                pltpu.sync_copy(row0c, acc_sh.at[ixd0], add=True)
                pltpu.sync_copy(ones_ch, deg_sh.at[ixd0], add=True)

                @pl.when(j0 + 2 < NCHUNK)
                def _():
                    start_idx(j0 + 2, ixs0, ixd0, semI0)
                    wait_idx(ixs0, ixd0, semI0)
                    pltpu.async_copy(h_hbm.at[ixs0], row0c, semG0)
                pltpu.make_async_copy(h_hbm.at[ixs1], row1, semG1).wait()
                pltpu.sync_copy(row1, acc_sh.at[ixd1], add=True)
                pltpu.sync_copy(ones_ch, deg_sh.at[ixd1], add=True)

                @pl.when(j1 + 2 < NCHUNK)
                def _():
                    start_idx(j1 + 2, ixs1, ixd1, semI1)
                    wait_idx(ixs1, ixd1, semI1)
                return 0
            lax.fori_loop(0, NCHUNK // 2, pair, 0)
            plsc.subcore_barrier()

            for i in range(ITER_R):
                k = i * NS + s

                @pl.when(k < NROWCH)
                def _():
                    rb = k * RCH
                    pltpu.sync_copy(acc_sh.at[pl.ds(rb, RCH)], row0.at[pl.ds(0, RCH)])
                    pltpu.sync_copy(deg_sh.at[pl.ds(rb, RCH)], smallbuf.at[pl.ds(0, RCH)])

                    def rbody(q, _):
                        dvec = smallbuf[pl.ds(q * LANES, LANES)]
                        rec = one16 / jnp.maximum(dvec, one16)
                        for rr in range(LANES):
                            r2 = q * LANES + rr
                            rec16 = jnp.broadcast_to(rec[rr], (LANES,))
                            for cc in range(NCC):
                                sl = pl.ds(cc * LANES, LANES)
                                row0[r2, sl] = row0[r2, sl] * rec16
                        return 0
                    lax.fori_loop(0, RCH // LANES, rbody, 0)
                    pltpu.sync_copy(row0.at[pl.ds(0, RCH)], out_hbm.at[pl.ds(g * N + rb, RCH)])
            plsc.subcore_barrier()

    return agg(h, src_flat, dst_flat)


def kernel(x, edge_index, W, b):
    B, N, Cin = x.shape
    Cout = W.shape[1]
    E = edge_index.shape[2]

    h = _project(x.reshape(B * N, Cin), W, b.reshape(1, Cout))

    offs = (jnp.arange(B, dtype=jnp.int32) * N)[:, None]
    src = (edge_index[:, 1, :] + offs).reshape(B * E)
    dst = edge_index[:, 0, :].reshape(B * E)

    out = _aggregate(h, src, dst, B=B, N=N, E=E, C=Cout)
    return out.reshape(B, N, Cout)


# speedup vs baseline: 2.7663x; 1.0512x over previous
import functools

import jax
import jax.numpy as jnp
from jax import lax
from jax.experimental import pallas as pl
from jax.experimental.pallas import tpu as pltpu
from jax.experimental.pallas import tpu_sc as plsc

NC = 2
NS = 16
LANES = 16


def _largest_div(total, hi, step):
    for c in range(hi, step - 1, -step):
        if total % c == 0:
            return c
    return None


def _project(x2, W, b2):
    M, Cin = x2.shape
    Cout = W.shape[1]
    BM = _largest_div(M, 2048, 8) or M

    def body(x_ref, w_ref, b_ref, o_ref):
        o_ref[...] = (
            jnp.dot(x_ref[...], w_ref[...], preferred_element_type=jnp.float32)
            + b_ref[...]
        )

    return pl.pallas_call(
        body,
        grid=(M // BM,),
        in_specs=[
            pl.BlockSpec((BM, Cin), lambda i: (i, 0)),
            pl.BlockSpec((Cin, Cout), lambda i: (0, 0)),
            pl.BlockSpec((1, Cout), lambda i: (0, 0)),
        ],
        out_specs=pl.BlockSpec((BM, Cout), lambda i: (i, 0)),
        out_shape=jax.ShapeDtypeStruct((M, Cout), jnp.float32),
    )(x2, W, b2)


def _aggregate(h, src_flat, dst_flat, B, N, E, C):
    assert B % NC == 0 and E % NS == 0 and C % LANES == 0
    ROUNDS = B // NC
    EPT = E // NS
    CH = _largest_div(EPT, 80, 8)
    assert CH is not None
    NCHUNK = EPT // CH
    assert NCHUNK % 2 == 0
    RCH = _largest_div(N, 128, 16)
    assert RCH is not None
    NROWCH = N // RCH
    ITER_R = -(-NROWCH // NS)
    NCC = C // LANES
    BUF = max(CH, RCH)

    mesh = plsc.VectorSubcoreMesh(core_axis_name="c", subcore_axis_name="s")

    @functools.partial(
        pl.kernel,
        mesh=mesh,
        out_type=jax.ShapeDtypeStruct((B * N, C), jnp.float32),
        scratch_types=[
            pltpu.VMEM((CH,), jnp.int32),
            pltpu.VMEM((CH,), jnp.int32),
            pltpu.VMEM((CH,), jnp.int32),
            pltpu.VMEM((CH,), jnp.int32),
            pltpu.VMEM((CH,), jnp.int32),
            pltpu.VMEM((CH,), jnp.int32),
            pltpu.VMEM((BUF, C), jnp.float32),
            pltpu.VMEM((CH, C), jnp.float32),
            pltpu.VMEM((CH, C), jnp.float32),
            pltpu.VMEM((BUF,), jnp.float32),
            pltpu.VMEM_SHARED((N, C), jnp.float32),
            pltpu.VMEM_SHARED((N,), jnp.float32),
            pltpu.SemaphoreType.DMA,
            pltpu.SemaphoreType.DMA,
            pltpu.SemaphoreType.DMA,
            pltpu.SemaphoreType.DMA,
            pltpu.SemaphoreType.DMA,
            pltpu.SemaphoreType.DMA,
        ],
    )
    def agg(h_hbm, src_hbm, dst_hbm, out_hbm,
            ixs0, ixd0, ixs1, ixd1, ixs2, ixd2, row0, row1, row2, smallbuf,
            acc_sh, deg_sh, semI0, semI1, semI2, semG0, semG1, semG2):
        c = lax.axis_index("c")
        s = lax.axis_index("s")

        one16 = jnp.full((LANES,), 1.0, jnp.float32)
        zero16 = jnp.zeros((LANES,), jnp.float32)
        row0c = row0.at[pl.ds(0, CH)]
        ones_ch = smallbuf.at[pl.ds(0, CH)]

        def fill_small(val):
            def fbody(i, _):
                smallbuf[pl.ds(i * LANES, LANES)] = val
                return 0
            lax.fori_loop(0, BUF // LANES, fbody, 0)

        for r in range(ROUNDS):
            g = r * NC + c

            fill_small(zero16)

            def zbody(i, _):
                for cc in range(NCC):
                    row0[i, pl.ds(cc * LANES, LANES)] = zero16
                return 0
            lax.fori_loop(0, BUF, zbody, 0)
            for i in range(ITER_R):
                k = i * NS + s

                @pl.when(k < NROWCH)
                def _():
                    rb = k * RCH
                    pltpu.sync_copy(row0.at[pl.ds(0, RCH)], acc_sh.at[pl.ds(rb, RCH)])
                    pltpu.sync_copy(smallbuf.at[pl.ds(0, RCH)], deg_sh.at[pl.ds(rb, RCH)])
            fill_small(one16)
            plsc.subcore_barrier()

            ebase = g * E + s * EPT

            def start_idx(j, ixs, ixd, semI):
                e0 = ebase + j * CH
                pltpu.async_copy(src_hbm.at[pl.ds(e0, CH)], ixs, semI)
                pltpu.async_copy(dst_hbm.at[pl.ds(e0, CH)], ixd, semI)

            def wait_idx(ixs, ixd, semI):
                pltpu.make_async_copy(src_hbm.at[pl.ds(ebase, CH)], ixs, semI).wait()
                pltpu.make_async_copy(dst_hbm.at[pl.ds(ebase, CH)], ixd, semI).wait()

            slots = ((ixs0, ixd0, row0c, semI0, semG0),
                     (ixs1, ixd1, row1, semI1, semG1),
                     (ixs2, ixd2, row2, semI2, semG2))
            NSLOT = len(slots)

            for k, (ixs, ixd, row, semI, semG) in enumerate(slots):
                start_idx(k, ixs, ixd, semI)
                wait_idx(ixs, ixd, semI)
                pltpu.async_copy(h_hbm.at[ixs], row, semG)

            def turn(p, _):
                jb = NSLOT * p
                for k, (ixs, ixd, row, semI, semG) in enumerate(slots):
                    j = jb + k

                    @pl.when(j < NCHUNK)
                    def _():
                        pltpu.make_async_copy(h_hbm.at[ixs], row, semG).wait()
                        pltpu.sync_copy(row, acc_sh.at[ixd], add=True)
                        pltpu.sync_copy(ones_ch, deg_sh.at[ixd], add=True)

                        @pl.when(j + NSLOT < NCHUNK)
                        def _():
                            start_idx(j + NSLOT, ixs, ixd, semI)
                            wait_idx(ixs, ixd, semI)
                            pltpu.async_copy(h_hbm.at[ixs], row, semG)
                return 0
            lax.fori_loop(0, -(-NCHUNK // NSLOT), turn, 0)
            plsc.subcore_barrier()

            for i in range(ITER_R):
                k = i * NS + s

                @pl.when(k < NROWCH)
                def _():
                    rb = k * RCH
                    pltpu.sync_copy(acc_sh.at[pl.ds(rb, RCH)], row0.at[pl.ds(0, RCH)])
                    pltpu.sync_copy(deg_sh.at[pl.ds(rb, RCH)], smallbuf.at[pl.ds(0, RCH)])

                    def rbody(q, _):
                        dvec = smallbuf[pl.ds(q * LANES, LANES)]
                        rec = one16 / jnp.maximum(dvec, one16)
                        for rr in range(LANES):
                            r2 = q * LANES + rr
                            rec16 = jnp.broadcast_to(rec[rr], (LANES,))
                            for cc in range(NCC):
                                sl = pl.ds(cc * LANES, LANES)
                                row0[r2, sl] = row0[r2, sl] * rec16
                        return 0
                    lax.fori_loop(0, RCH // LANES, rbody, 0)
                    pltpu.sync_copy(row0.at[pl.ds(0, RCH)], out_hbm.at[pl.ds(g * N + rb, RCH)])
            plsc.subcore_barrier()

    return agg(h, src_flat, dst_flat)


def kernel(x, edge_index, W, b):
    B, N, Cin = x.shape
    Cout = W.shape[1]
    E = edge_index.shape[2]

    h = _project(x.reshape(B * N, Cin), W, b.reshape(1, Cout))

    offs = (jnp.arange(B, dtype=jnp.int32) * N)[:, None]
    src = (edge_index[:, 1, :] + offs).reshape(B * E)
    dst = edge_index[:, 0, :].reshape(B * E)

    out = _aggregate(h, src, dst, B=B, N=N, E=E, C=Cout)
    return out.reshape(B, N, Cout)


# speedup vs baseline: 2.7682x; 1.0007x over previous
import functools

import jax
import jax.numpy as jnp
from jax import lax
from jax.experimental import pallas as pl
from jax.experimental.pallas import tpu as pltpu
from jax.experimental.pallas import tpu_sc as plsc

NC = 2
NS = 16
LANES = 16


def _largest_div(total, hi, step):
    for c in range(hi, step - 1, -step):
        if total % c == 0:
            return c
    return None


def _project(x2, W, b2):
    M, Cin = x2.shape
    Cout = W.shape[1]
    BM = _largest_div(M, 2048, 8) or M

    def body(x_ref, w_ref, b_ref, o_ref):
        o_ref[...] = (
            jnp.dot(x_ref[...], w_ref[...], preferred_element_type=jnp.float32)
            + b_ref[...]
        )

    return pl.pallas_call(
        body,
        grid=(M // BM,),
        in_specs=[
            pl.BlockSpec((BM, Cin), lambda i: (i, 0)),
            pl.BlockSpec((Cin, Cout), lambda i: (0, 0)),
            pl.BlockSpec((1, Cout), lambda i: (0, 0)),
        ],
        out_specs=pl.BlockSpec((BM, Cout), lambda i: (i, 0)),
        out_shape=jax.ShapeDtypeStruct((M, Cout), jnp.float32),
    )(x2, W, b2)


def _aggregate(h, src_flat, dst_flat, B, N, E, C):
    assert B % NC == 0 and E % NS == 0 and C % LANES == 0
    ROUNDS = B // NC
    EPT = E // NS
    CH = _largest_div(EPT, 80, 8)
    assert CH is not None
    NCHUNK = EPT // CH
    RCH = _largest_div(N, 128, 16)
    assert RCH is not None
    NROWCH = N // RCH
    ITER_R = -(-NROWCH // NS)
    NCC = C // LANES
    BUF = max(CH, RCH)

    mesh = plsc.VectorSubcoreMesh(core_axis_name="c", subcore_axis_name="s")

    @functools.partial(
        pl.kernel,
        mesh=mesh,
        out_type=jax.ShapeDtypeStruct((B * N, C), jnp.float32),
        scratch_types=[
            pltpu.VMEM((CH,), jnp.int32),
            pltpu.VMEM((CH,), jnp.int32),
            pltpu.VMEM((CH,), jnp.int32),
            pltpu.VMEM((CH,), jnp.int32),
            pltpu.VMEM((CH,), jnp.int32),
            pltpu.VMEM((CH,), jnp.int32),
            pltpu.VMEM((BUF, C), jnp.float32),
            pltpu.VMEM((CH, C), jnp.float32),
            pltpu.VMEM((CH, C), jnp.float32),
            pltpu.VMEM((BUF,), jnp.float32),
            pltpu.VMEM_SHARED((N, C), jnp.float32),
            pltpu.VMEM_SHARED((N,), jnp.float32),
            pltpu.SemaphoreType.DMA,
            pltpu.SemaphoreType.DMA,
            pltpu.SemaphoreType.DMA,
            pltpu.SemaphoreType.DMA,
            pltpu.SemaphoreType.DMA,
            pltpu.SemaphoreType.DMA,
        ],
    )
    def agg(h_hbm, src_hbm, dst_hbm, out_hbm,
            ixs0, ixd0, ixs1, ixd1, ixs2, ixd2, row0, row1, row2, smallbuf,
            acc_sh, deg_sh, semI0, semI1, semI2, semG0, semG1, semG2):
        c = lax.axis_index("c")
        s = lax.axis_index("s")

        one16 = jnp.full((LANES,), 1.0, jnp.float32)
        zero16 = jnp.zeros((LANES,), jnp.float32)
        row0c = row0.at[pl.ds(0, CH)]
        ones_ch = smallbuf.at[pl.ds(0, CH)]

        def fill_small(val):
            def fbody(i, _):
                smallbuf[pl.ds(i * LANES, LANES)] = val
                return 0
            lax.fori_loop(0, BUF // LANES, fbody, 0)

        for r in range(ROUNDS):
            g = r * NC + c

            fill_small(zero16)

            def zbody(i, _):
                for cc in range(NCC):
                    row0[i, pl.ds(cc * LANES, LANES)] = zero16
                return 0
            lax.fori_loop(0, BUF, zbody, 0)
            for i in range(ITER_R):
                k = i * NS + s

                @pl.when(k < NROWCH)
                def _():
                    rb = k * RCH
                    pltpu.sync_copy(row0.at[pl.ds(0, RCH)], acc_sh.at[pl.ds(rb, RCH)])
                    pltpu.sync_copy(smallbuf.at[pl.ds(0, RCH)], deg_sh.at[pl.ds(rb, RCH)])
            fill_small(one16)
            plsc.subcore_barrier()

            ebase = g * E + s * EPT

            def start_idx(j, ixs, ixd, semI):
                e0 = ebase + j * CH
                pltpu.async_copy(src_hbm.at[pl.ds(e0, CH)], ixs, semI)
                pltpu.async_copy(dst_hbm.at[pl.ds(e0, CH)], ixd, semI)

            def wait_idx(ixs, ixd, semI):
                pltpu.make_async_copy(src_hbm.at[pl.ds(ebase, CH)], ixs, semI).wait()
                pltpu.make_async_copy(dst_hbm.at[pl.ds(ebase, CH)], ixd, semI).wait()

            slots = ((ixs0, ixd0, row0c, semI0, semG0),
                     (ixs1, ixd1, row1, semI1, semG1),
                     (ixs2, ixd2, row2, semI2, semG2))
            NSLOT = len(slots)

            for k, (ixs, ixd, row, semI, semG) in enumerate(slots):
                start_idx(k, ixs, ixd, semI)
                wait_idx(ixs, ixd, semI)
                pltpu.async_copy(h_hbm.at[ixs], row, semG)

            def turn(p, _):
                jb = NSLOT * p
                for k, (ixs, ixd, row, semI, semG) in enumerate(slots):
                    j = jb + k

                    @pl.when(j < NCHUNK)
                    def _():
                        pltpu.make_async_copy(h_hbm.at[ixs], row, semG).wait()
                        pltpu.sync_copy(row, acc_sh.at[ixd], add=True)
                        pltpu.sync_copy(ones_ch, deg_sh.at[ixd], add=True)

                        @pl.when(j + NSLOT < NCHUNK)
                        def _():
                            start_idx(j + NSLOT, ixs, ixd, semI)
                            wait_idx(ixs, ixd, semI)
                            pltpu.async_copy(h_hbm.at[ixs], row, semG)
                return 0
            lax.fori_loop(0, -(-NCHUNK // NSLOT), turn, 0)
            plsc.subcore_barrier()

            for i in range(ITER_R):
                k = i * NS + s

                @pl.when(k < NROWCH)
                def _():
                    rb = k * RCH
                    pltpu.sync_copy(acc_sh.at[pl.ds(rb, RCH)], row0.at[pl.ds(0, RCH)])
                    pltpu.sync_copy(deg_sh.at[pl.ds(rb, RCH)], smallbuf.at[pl.ds(0, RCH)])

                    def rbody(q, _):
                        dvec = smallbuf[pl.ds(q * LANES, LANES)]
                        rec = one16 / jnp.maximum(dvec, one16)
                        for rr in range(LANES):
                            r2 = q * LANES + rr
                            rec16 = jnp.broadcast_to(rec[rr], (LANES,))
                            for cc in range(NCC):
                                sl = pl.ds(cc * LANES, LANES)
                                row0[r2, sl] = row0[r2, sl] * rec16
                        return 0
                    lax.fori_loop(0, RCH // LANES, rbody, 0)
                    pltpu.sync_copy(row0.at[pl.ds(0, RCH)], out_hbm.at[pl.ds(g * N + rb, RCH)])
            plsc.subcore_barrier()

    return agg(h, src_flat, dst_flat)


def kernel(x, edge_index, W, b):
    B, N, Cin = x.shape
    Cout = W.shape[1]
    E = edge_index.shape[2]

    h = _project(x.reshape(B * N, Cin), W, b.reshape(1, Cout))

    offs = (jnp.arange(B, dtype=jnp.int32) * N)[:, None]
    src = (edge_index[:, 1, :] + offs).reshape(B * E)
    dst = edge_index[:, 0, :].reshape(B * E)

    out = _aggregate(h, src, dst, B=B, N=N, E=E, C=Cout)
    return out.reshape(B, N, Cout)
